# baseline (speedup 1.0000x reference)
import jax
import jax.numpy as jnp
from jax.experimental import pallas as pl
from jax.experimental.pallas import tpu as pltpu


def _wmae_body(w_ref, e_ref, y0_ref, y1_ref, p0_ref, p1_ref, out_ref):
    j = pl.program_id(1)

    nb = y0_ref.shape[0]
    hh = y0_ref.shape[2]
    wd_ = y0_ref.shape[3]
    rows = 32
    acc_s = jnp.zeros((8, wd_), jnp.float32)
    acc_c = jnp.zeros((8, wd_), jnp.float32)
    for y_ref, p_ref in ((y0_ref, p0_ref), (y1_ref, p1_ref)):
        for k in range(nb):
            for c in range(hh // rows):
                y = y_ref[k, 0, c * rows:(c + 1) * rows, :]
                d = jnp.abs(p_ref[k, 0, c * rows:(c + 1) * rows, :] - y)
                w = jnp.zeros_like(y)
                for b in range(w_ref.shape[0]):
                    w = jnp.where(y >= e_ref[b], w_ref[b], w)
                below_top = y < e_ref[e_ref.shape[0] - 1]
                w = jnp.where(below_top, w, 0.0)
                valid = jnp.where((y >= e_ref[0]) & below_top, 1.0, 0.0)
                wd = w * d
                for r in range(rows // 8):
                    acc_s = acc_s + wd[r * 8:(r + 1) * 8, :]
                    acc_c = acc_c + valid[r * 8:(r + 1) * 8, :]
    ps = jnp.sum(acc_s)
    pc = jnp.sum(acc_c)

    @pl.when(j == 0)
    def _():
        out_ref[0, 0, 0] = ps
        out_ref[0, 0, 1] = pc

    @pl.when(j > 0)
    def _():
        out_ref[0, 0, 0] += ps
        out_ref[0, 0, 1] += pc


def kernel(y, p, weights, edge):
    b, h, w = y.shape
    y4 = y.reshape(b, 2, h // 2, w)
    p4 = p.reshape(b, 2, h // 2, w)
    ncores = 2
    bb = 4
    per = b // (ncores * bb)
    blk = (bb, 1, h // 2, w)
    partials = pl.pallas_call(
        _wmae_body,
        grid=(ncores, per),
        in_specs=[
            pl.BlockSpec(memory_space=pltpu.SMEM),
            pl.BlockSpec(memory_space=pltpu.SMEM),
            pl.BlockSpec(blk, lambda i, j: (j * 2 + i, 0, 0, 0)),
            pl.BlockSpec(blk, lambda i, j: (j * 2 + i, 1, 0, 0)),
            pl.BlockSpec(blk, lambda i, j: (j * 2 + i, 0, 0, 0)),
            pl.BlockSpec(blk, lambda i, j: (j * 2 + i, 1, 0, 0)),
        ],
        out_specs=pl.BlockSpec((1, 1, 2), lambda i, j: (i, 0, 0),
                               memory_space=pltpu.SMEM),
        out_shape=jax.ShapeDtypeStruct((ncores, 1, 2), jnp.float32),
        compiler_params=pltpu.CompilerParams(
            dimension_semantics=("parallel", "arbitrary")),
    )(weights, edge, y4, y4, p4, p4)
    return partials[:, 0, 0].sum() / partials[:, 0, 1].sum()


# speedup vs baseline: 1.0067x; 1.0067x over previous
import jax
import jax.numpy as jnp
from jax.experimental import pallas as pl
from jax.experimental.pallas import tpu as pltpu


def _wmae_body(w_ref, e_ref, y0_ref, y1_ref, p0_ref, p1_ref, out_ref):
    j = pl.program_id(1)

    nb = y0_ref.shape[0]
    hh = y0_ref.shape[2]
    wd_ = y0_ref.shape[3]
    rows = 32
    acc_s = jnp.zeros((8, wd_), jnp.float32)
    acc_c = jnp.zeros((8, wd_), jnp.float32)
    for y_ref, p_ref in ((y0_ref, p0_ref), (y1_ref, p1_ref)):
        for k in range(nb):
            for c in range(hh // rows):
                y = y_ref[k, 0, c * rows:(c + 1) * rows, :]
                d = jnp.abs(p_ref[k, 0, c * rows:(c + 1) * rows, :] - y)
                w = jnp.zeros_like(y)
                for b in range(w_ref.shape[0]):
                    w = jnp.where(y >= e_ref[b], w_ref[b], w)
                below_top = y < e_ref[e_ref.shape[0] - 1]
                w = jnp.where(below_top, w, 0.0)
                valid = jnp.where((y >= e_ref[0]) & below_top, 1.0, 0.0)
                wd = w * d
                for r in range(rows // 8):
                    acc_s = acc_s + wd[r * 8:(r + 1) * 8, :]
                    acc_c = acc_c + valid[r * 8:(r + 1) * 8, :]
    ps = jnp.sum(acc_s)
    pc = jnp.sum(acc_c)

    @pl.when(j == 0)
    def _():
        out_ref[0, 0, 0] = ps
        out_ref[0, 0, 1] = pc

    @pl.when(j > 0)
    def _():
        out_ref[0, 0, 0] += ps
        out_ref[0, 0, 1] += pc


def kernel(y, p, weights, edge):
    b, h, w = y.shape
    y4 = y.reshape(b, 2, h // 2, w)
    p4 = p.reshape(b, 2, h // 2, w)
    ncores = 1
    bb = 4
    per = b // (ncores * bb)
    blk = (bb, 1, h // 2, w)
    partials = pl.pallas_call(
        _wmae_body,
        grid=(ncores, per),
        in_specs=[
            pl.BlockSpec(memory_space=pltpu.SMEM),
            pl.BlockSpec(memory_space=pltpu.SMEM),
            pl.BlockSpec(blk, lambda i, j: (j * 2 + i, 0, 0, 0)),
            pl.BlockSpec(blk, lambda i, j: (j * 2 + i, 1, 0, 0)),
            pl.BlockSpec(blk, lambda i, j: (j * 2 + i, 0, 0, 0)),
            pl.BlockSpec(blk, lambda i, j: (j * 2 + i, 1, 0, 0)),
        ],
        out_specs=pl.BlockSpec((1, 1, 2), lambda i, j: (i, 0, 0),
                               memory_space=pltpu.SMEM),
        out_shape=jax.ShapeDtypeStruct((ncores, 1, 2), jnp.float32),
        compiler_params=pltpu.CompilerParams(
            dimension_semantics=("parallel", "arbitrary")),
    )(weights, edge, y4, y4, p4, p4)
    return partials[:, 0, 0].sum() / partials[:, 0, 1].sum()
